# baseline (speedup 1.0000x reference)
import functools

import jax
import jax.numpy as jnp
import numpy as np
from jax import lax
from jax.experimental import pallas as pl
from jax.experimental.pallas import tpu as pltpu
from jax.experimental.pallas import tpu_sc as plsc

B = 4
S = 4096
D = 1024
H = 512
K = 40
KPAD = 64
MMBLK = 4096
BLK = 512
NBLK = (B * S) // BLK
ROWS_PER_B = S // BLK

_ERFC_P = [2.326819970068386e-2, -1.387039388740657e-1, 3.687424674597105e-1,
           -5.824733027278666e-1, 6.210004621745983e-1, -4.944515323274145e-1,
           3.404879937665872e-1, -2.741127028184656e-1, 5.638259427386472e-1]
_ERFC_R = [-1.047766399936249e+1, 1.297719955372516e+1, -7.495518717768503e+0,
           2.921019019210786e+0, -1.015265279202700e+0, 4.218463358204948e-1,
           -2.820767439740514e-1, 5.641895067754075e-1]
_ERF_T = [7.853861353153693e-5, -8.010193625184903e-4, 5.188327685732524e-3,
          -2.685381193529856e-2, 1.128358514861418e-1, -3.761262582423300e-1,
          1.128379165726710e+0]


def _poly(y, coefs):
    p = jnp.full_like(y, np.float32(coefs[0]))
    for c in coefs[1:]:
        p = p * y + np.float32(c)
    return p


def _erfc(x):
    abs_x = jnp.abs(x)
    z = jnp.exp(-x * x)
    q = 1.0 / abs_x
    y = q * q
    p = jnp.where(abs_x < 2.0, _poly(y, _ERFC_P), _poly(y, _ERFC_R))
    yv = z * q * p
    y_clamp = jnp.where(z < 1e-38, 0.0, yv)
    big = jnp.where(x < 0.0, 2.0 - y_clamp, y_clamp)
    small = 1.0 - x * _poly(x * x, _ERF_T)
    return jnp.where(abs_x > 1.0, big, small)


def _gelu(x):
    return (0.5 * x) * _erfc(-x * np.float32(np.sqrt(0.5)))


def _mm_body(x_ref, w_ref, o_ref):
    o_ref[...] = lax.dot_general(x_ref[...], w_ref[...],
                                 (((1,), (0,)), ((), ())),
                                 preferred_element_type=jnp.float32)


_mm = pl.pallas_call(
    _mm_body,
    grid=((B * S) // MMBLK,),
    in_specs=[pl.BlockSpec((MMBLK, D), lambda i: (i, 0)),
              pl.BlockSpec((D, H), lambda i: (0, 0))],
    out_specs=pl.BlockSpec((MMBLK, H), lambda i: (i, 0)),
    out_shape=jax.ShapeDtypeStruct((B * S, H), jnp.float32),
)


def _score_topk_body(h_ref, mu_ref, var_ref, b1_ref, g_ref, be_ref, w2_ref,
                     b2_ref, idx_ref, pad_ref, s_scratch):
    i = pl.program_id(0)
    h = h_ref[...] + b1_ref[...]
    hn = ((h - mu_ref[...]) * lax.rsqrt(var_ref[...] + 1e-5)) * g_ref[...] \
        + be_ref[...]
    ge = _gelu(hn)
    z = lax.dot_general(ge, w2_ref[...], (((1,), (0,)), ((), ())),
                        preferred_element_type=jnp.float32)
    sc = jax.nn.sigmoid(z + b2_ref[0, 0])
    s_scratch[pl.ds(i, 1), :] = sc.reshape(1, BLK)

    @pl.when(i == NBLK - 1)
    def _topk():
        lane64 = lax.broadcasted_iota(jnp.int32, (1, KPAD), 1)
        for b in range(B):
            sb = s_scratch[b * ROWS_PER_B:(b + 1) * ROWS_PER_B, :]
            iota = (lax.broadcasted_iota(jnp.int32, (ROWS_PER_B, BLK), 0) * BLK
                    + lax.broadcasted_iota(jnp.int32, (ROWS_PER_B, BLK), 1))

            def body(kk, carry):
                s, acc = carry
                m = jnp.max(s)
                idx = jnp.min(jnp.where(s == m, iota, S))
                acc = jnp.where(lane64 == kk, idx, acc)
                s = jnp.where(iota == idx, -jnp.inf, s)
                return s, acc

            _, acc = lax.fori_loop(
                0, K, body, (sb, jnp.zeros((1, KPAD), jnp.int32)))
            idx_ref[pl.ds(b, 1), :] = acc[:, :K]
            pad_ref[pl.ds(b, 1), :] = acc + b * S


_score_topk = pl.pallas_call(
    _score_topk_body,
    grid=(NBLK,),
    in_specs=[
        pl.BlockSpec((BLK, H), lambda i: (i, 0)),
        pl.BlockSpec((BLK, 1), lambda i: (i, 0)),
        pl.BlockSpec((BLK, 1), lambda i: (i, 0)),
        pl.BlockSpec((1, H), lambda i: (0, 0)),
        pl.BlockSpec((1, H), lambda i: (0, 0)),
        pl.BlockSpec((1, H), lambda i: (0, 0)),
        pl.BlockSpec((H, 1), lambda i: (0, 0)),
        pl.BlockSpec((1, 1), lambda i: (0, 0)),
    ],
    out_specs=[
        pl.BlockSpec((B, K), lambda i: (0, 0)),
        pl.BlockSpec((B, KPAD), lambda i: (0, 0)),
    ],
    out_shape=[
        jax.ShapeDtypeStruct((B, K), jnp.int32),
        jax.ShapeDtypeStruct((B, KPAD), jnp.int32),
    ],
    scratch_shapes=[pltpu.VMEM((NBLK, BLK), jnp.float32)],
)


def _make_gather():
    nc, ns = 2, 16
    nw = nc * ns
    rows_per_w = (B * KPAD) // nw
    mesh = plsc.VectorSubcoreMesh(
        core_axis_name="c", subcore_axis_name="s",
        num_cores=nc, num_subcores=ns)

    @functools.partial(
        pl.kernel, mesh=mesh,
        out_type=jax.ShapeDtypeStruct((B * KPAD, D), jnp.float32),
        scratch_types=[
            pltpu.VMEM((rows_per_w,), jnp.int32),
            pltpu.VMEM((rows_per_w, D), jnp.float32),
            pltpu.SemaphoreType.DMA,
        ],
    )
    def gather(table_hbm, idx_hbm, out_hbm, idx_v, rows_v, sem):
        wid = lax.axis_index("s") * nc + lax.axis_index("c")
        base = wid * rows_per_w
        pltpu.sync_copy(idx_hbm.at[pl.ds(base, rows_per_w)], idx_v)
        pltpu.async_copy(table_hbm.at[idx_v], rows_v, sem).wait()
        pltpu.sync_copy(rows_v, out_hbm.at[pl.ds(base, rows_per_w)])

    return gather


_gather_cache = []


def _gather(table, idx):
    if not _gather_cache:
        _gather_cache.append(_make_gather())
    return _gather_cache[0](table, idx)


def kernel(features, W1, b1, gamma, beta, W2, b2):
    feat2d = features.reshape(B * S, D)
    h = _mm(feat2d, W1)
    mu = jnp.mean(h, axis=-1, keepdims=True)
    var = jnp.var(h, axis=-1, keepdims=True)
    indices, pad = _score_topk(
        h, mu, var, b1.reshape(1, H), gamma.reshape(1, H),
        beta.reshape(1, H), W2, b2.reshape(1, 1))
    rows = _gather(feat2d, pad.reshape(-1))
    tokens = rows.reshape(B, KPAD, D)[:, :K, :]
    return tokens, indices


# speedup vs baseline: 1.0618x; 1.0618x over previous
import functools

import jax
import jax.numpy as jnp
import numpy as np
from jax import lax
from jax.experimental import pallas as pl
from jax.experimental.pallas import tpu as pltpu
from jax.experimental.pallas import tpu_sc as plsc

B = 4
S = 4096
D = 1024
H = 512
K = 40
KPAD = 64
MMBLK = 4096
BLK = 512
NBLK = (B * S) // BLK
ROWS_PER_B = S // BLK

_ERFC_P = [2.326819970068386e-2, -1.387039388740657e-1, 3.687424674597105e-1,
           -5.824733027278666e-1, 6.210004621745983e-1, -4.944515323274145e-1,
           3.404879937665872e-1, -2.741127028184656e-1, 5.638259427386472e-1]
_ERFC_R = [-1.047766399936249e+1, 1.297719955372516e+1, -7.495518717768503e+0,
           2.921019019210786e+0, -1.015265279202700e+0, 4.218463358204948e-1,
           -2.820767439740514e-1, 5.641895067754075e-1]
_ERF_T = [7.853861353153693e-5, -8.010193625184903e-4, 5.188327685732524e-3,
          -2.685381193529856e-2, 1.128358514861418e-1, -3.761262582423300e-1,
          1.128379165726710e+0]


def _poly(y, coefs):
    p = jnp.full_like(y, np.float32(coefs[0]))
    for c in coefs[1:]:
        p = p * y + np.float32(c)
    return p


def _erfc(x):
    abs_x = jnp.abs(x)
    z = jnp.exp(-x * x)
    q = 1.0 / abs_x
    y = q * q
    p = jnp.where(abs_x < 2.0, _poly(y, _ERFC_P), _poly(y, _ERFC_R))
    yv = z * q * p
    y_clamp = jnp.where(z < 1e-38, 0.0, yv)
    big = jnp.where(x < 0.0, 2.0 - y_clamp, y_clamp)
    small = 1.0 - x * _poly(x * x, _ERF_T)
    return jnp.where(abs_x > 1.0, big, small)


def _gelu(x):
    return (0.5 * x) * _erfc(-x * np.float32(np.sqrt(0.5)))


def _mm_body(x_ref, w_ref, o_ref):
    o_ref[...] = lax.dot_general(x_ref[...], w_ref[...],
                                 (((1,), (0,)), ((), ())),
                                 preferred_element_type=jnp.float32)


_mm = pl.pallas_call(
    _mm_body,
    grid=((B * S) // MMBLK,),
    in_specs=[pl.BlockSpec((MMBLK, D), lambda i: (i, 0)),
              pl.BlockSpec((D, H), lambda i: (0, 0))],
    out_specs=pl.BlockSpec((MMBLK, H), lambda i: (i, 0)),
    out_shape=jax.ShapeDtypeStruct((B * S, H), jnp.float32),
)


def _score_topk_body(h_ref, mu_ref, var_ref, b1_ref, g_ref, be_ref, w2_ref,
                     b2_ref, idx_ref, pad_ref, s_scratch):
    i = pl.program_id(0)
    h = h_ref[...] + b1_ref[...]
    hn = ((h - mu_ref[...]) * lax.rsqrt(var_ref[...] + 1e-5)) * g_ref[...] \
        + be_ref[...]
    ge = _gelu(hn)
    z = lax.dot_general(ge, w2_ref[...], (((1,), (0,)), ((), ())),
                        preferred_element_type=jnp.float32)
    sc = jax.nn.sigmoid(z + b2_ref[0, 0])
    s_scratch[pl.ds(i, 1), :] = sc.reshape(1, BLK)

    @pl.when(i == NBLK - 1)
    def _topk():
        lane64 = lax.broadcasted_iota(jnp.int32, (1, KPAD), 1)
        iota = (lax.broadcasted_iota(jnp.int32, (ROWS_PER_B, BLK), 0) * BLK
                + lax.broadcasted_iota(jnp.int32, (ROWS_PER_B, BLK), 1))
        sbs = tuple(s_scratch[b * ROWS_PER_B:(b + 1) * ROWS_PER_B, :]
                    for b in range(B))
        accs = tuple(jnp.zeros((1, KPAD), jnp.int32) for _ in range(B))

        def body(kk, carry):
            sbs, accs = carry
            new_s, new_a = [], []
            for b in range(B):
                s, acc = sbs[b], accs[b]
                m = jnp.max(s)
                idx = jnp.min(jnp.where(s == m, iota, S))
                new_a.append(jnp.where(lane64 == kk, idx, acc))
                new_s.append(jnp.where(iota == idx, -jnp.inf, s))
            return tuple(new_s), tuple(new_a)

        _, accs = lax.fori_loop(0, K, body, (sbs, accs))
        for b in range(B):
            idx_ref[pl.ds(b, 1), :] = accs[b][:, :K]
            pad_ref[pl.ds(b, 1), :] = accs[b] + b * S


_score_topk = pl.pallas_call(
    _score_topk_body,
    grid=(NBLK,),
    in_specs=[
        pl.BlockSpec((BLK, H), lambda i: (i, 0)),
        pl.BlockSpec((BLK, 1), lambda i: (i, 0)),
        pl.BlockSpec((BLK, 1), lambda i: (i, 0)),
        pl.BlockSpec((1, H), lambda i: (0, 0)),
        pl.BlockSpec((1, H), lambda i: (0, 0)),
        pl.BlockSpec((1, H), lambda i: (0, 0)),
        pl.BlockSpec((H, 1), lambda i: (0, 0)),
        pl.BlockSpec((1, 1), lambda i: (0, 0)),
    ],
    out_specs=[
        pl.BlockSpec((B, K), lambda i: (0, 0)),
        pl.BlockSpec((B, KPAD), lambda i: (0, 0)),
    ],
    out_shape=[
        jax.ShapeDtypeStruct((B, K), jnp.int32),
        jax.ShapeDtypeStruct((B, KPAD), jnp.int32),
    ],
    scratch_shapes=[pltpu.VMEM((NBLK, BLK), jnp.float32)],
)


def _make_gather():
    nc, ns = 2, 16
    nw = nc * ns
    rows_per_w = (B * KPAD) // nw
    mesh = plsc.VectorSubcoreMesh(
        core_axis_name="c", subcore_axis_name="s",
        num_cores=nc, num_subcores=ns)

    @functools.partial(
        pl.kernel, mesh=mesh,
        out_type=jax.ShapeDtypeStruct((B * KPAD, D), jnp.float32),
        scratch_types=[
            pltpu.VMEM((rows_per_w,), jnp.int32),
            pltpu.VMEM((rows_per_w, D), jnp.float32),
            pltpu.SemaphoreType.DMA,
        ],
    )
    def gather(table_hbm, idx_hbm, out_hbm, idx_v, rows_v, sem):
        wid = lax.axis_index("s") * nc + lax.axis_index("c")
        base = wid * rows_per_w
        pltpu.sync_copy(idx_hbm.at[pl.ds(base, rows_per_w)], idx_v)
        pltpu.async_copy(table_hbm.at[idx_v], rows_v, sem).wait()
        pltpu.sync_copy(rows_v, out_hbm.at[pl.ds(base, rows_per_w)])

    return gather


_gather_cache = []


def _gather(table, idx):
    if not _gather_cache:
        _gather_cache.append(_make_gather())
    return _gather_cache[0](table, idx)


def kernel(features, W1, b1, gamma, beta, W2, b2):
    feat2d = features.reshape(B * S, D)
    h = _mm(feat2d, W1)
    mu = jnp.mean(h, axis=-1, keepdims=True)
    var = jnp.var(h, axis=-1, keepdims=True)
    indices, pad = _score_topk(
        h, mu, var, b1.reshape(1, H), gamma.reshape(1, H),
        beta.reshape(1, H), W2, b2.reshape(1, 1))
    rows = _gather(feat2d, pad.reshape(-1))
    tokens = rows.reshape(B, KPAD, D)[:, :K, :]
    return tokens, indices
